# baseline (speedup 1.0000x reference)
import functools

import jax
import jax.numpy as jnp
import numpy as np
from jax import lax
from jax.experimental import pallas as pl
from jax.experimental.pallas import tpu as pltpu
from jax.experimental.pallas import tpu_sc as plsc

_RCHUNK = 4
_CCHUNK = 4
_NCHUNKS = _RCHUNK * _CCHUNK

_IDXS_NP = np.array(
    [[1, 3, 9, 11, 5, 15, 0, 14, 2, 12, 6, 7, 13, 10, 4, 8],
     [2, 15, 10, 0, 4, 11, 12, 5, 7, 9, 13, 6, 3, 14, 1, 8],
     [5, 7, 10, 0, 1, 4, 2, 13, 12, 6, 3, 8, 11, 14, 15, 9],
     [15, 4, 5, 3, 2, 10, 11, 12, 7, 6, 0, 14, 13, 1, 9, 8],
     [6, 15, 13, 5, 11, 1, 9, 3, 2, 14, 7, 10, 8, 4, 12, 0],
     [8, 3, 1, 9, 13, 7, 12, 15, 2, 4, 0, 10, 11, 5, 6, 14],
     [11, 1, 8, 13, 7, 6, 14, 0, 10, 15, 5, 3, 12, 4, 9, 2],
     [5, 9, 13, 0, 2, 11, 10, 14, 8, 7, 1, 3, 4, 15, 6, 12],
     [11, 2, 12, 8, 3, 10, 13, 5, 4, 15, 0, 9, 14, 7, 6, 1],
     [2, 9, 11, 6, 8, 4, 7, 13, 15, 1, 5, 3, 0, 14, 12, 10],
     [5, 8, 6, 4, 12, 11, 14, 3, 0, 2, 1, 9, 7, 15, 10, 13],
     [15, 8, 9, 2, 11, 7, 14, 12, 0, 6, 1, 3, 13, 10, 4, 5],
     [1, 0, 13, 5, 14, 2, 10, 9, 15, 11, 8, 3, 6, 7, 4, 12],
     [12, 13, 9, 15, 6, 10, 3, 8, 0, 5, 7, 4, 14, 11, 2, 1],
     [0, 12, 5, 10, 15, 11, 9, 2, 1, 7, 4, 3, 6, 14, 8, 13],
     [13, 2, 8, 6, 3, 10, 0, 9, 7, 11, 4, 14, 12, 15, 1, 5]],
    dtype=np.int32,
)

_SRC = [
    int(_IDXS_NP[_CCHUNK * ((p := k // _CCHUNK) % _RCHUNK) + k % _CCHUNK, p // _RCHUNK])
    for k in range(16 * _RCHUNK * _CCHUNK)
]

_NWORK = 32
_PER_W = (16 * _RCHUNK) // _NWORK


_SLAB = 4


def _sc_body(protos_hbm, out_hbm, src_smem, buf0, buf1, in_sem, out_sem):
    N, C, H, W = protos_hbm.shape
    Hc, Wc = H // _RCHUNK, W // _CCHUNK
    nslab = C // _SLAB
    bufs = (buf0, buf1)
    for k in range(16 * _RCHUNK * _CCHUNK):
        src_smem[k] = np.int32(_SRC[k])
    wid = lax.axis_index("s") * 2 + lax.axis_index("c")

    def pair_body(q):
        p = wid * _PER_W + q
        n = lax.div(p, _RCHUNK)
        r = lax.rem(p, _RCHUNK)
        ho = r * Hc

        def fetch(t):
            co = t * _SLAB
            b = bufs[t % 2]
            ds = []
            for c in range(_CCHUNK):
                m = src_smem[p * _CCHUNK + c]
                d = pltpu.make_async_copy(
                    protos_hbm.at[m, pl.ds(co, _SLAB), pl.ds(ho, Hc),
                                  pl.ds(c * Wc, Wc)],
                    b.at[:, :, pl.ds(c * Wc, Wc)],
                    in_sem,
                )
                d.start()
                ds.append(d)
            return ds

        def wb(t):
            co = t * _SLAB
            d = pltpu.make_async_copy(
                bufs[t % 2], out_hbm.at[n, pl.ds(co, _SLAB), pl.ds(ho, Hc), :],
                out_sem,
            )
            d.start()
            return d

        in_d = [None] * nslab
        out_d = [None] * nslab
        in_d[0] = fetch(0)
        for t in range(nslab):
            if t + 1 < nslab:
                if t >= 1:
                    out_d[t - 1].wait()
                in_d[t + 1] = fetch(t + 1)
            for d in in_d[t]:
                d.wait()
            out_d[t] = wb(t)
        out_d[nslab - 2].wait()
        out_d[nslab - 1].wait()

    pl.loop(0, _PER_W)(pair_body)


def kernel(protos):
    N, C, H, W = protos.shape
    Hc = H // _RCHUNK
    mesh = plsc.VectorSubcoreMesh(core_axis_name="c", subcore_axis_name="s")
    spro = pl.kernel(
        _sc_body,
        out_type=jax.ShapeDtypeStruct((N, C, H, W), protos.dtype),
        mesh=mesh,
        scratch_types=[
            pltpu.SMEM((16 * _RCHUNK * _CCHUNK,), jnp.int32),
            pltpu.VMEM((_SLAB, Hc, W), jnp.float32),
            pltpu.VMEM((_SLAB, Hc, W), jnp.float32),
            pltpu.SemaphoreType.DMA,
            pltpu.SemaphoreType.DMA,
        ],
        compiler_params=pltpu.CompilerParams(use_tc_tiling_on_sc=False),
    )(protos)
    return spro, jnp.asarray(_IDXS_NP)


# speedup vs baseline: 4.9499x; 4.9499x over previous
import jax
import jax.numpy as jnp
import numpy as np
from jax.experimental import pallas as pl
from jax.experimental.pallas import tpu as pltpu

_RCHUNK = 4
_CCHUNK = 4
_NCHUNKS = _RCHUNK * _CCHUNK

_IDXS_NP = np.array(
    [[1, 3, 9, 11, 5, 15, 0, 14, 2, 12, 6, 7, 13, 10, 4, 8],
     [2, 15, 10, 0, 4, 11, 12, 5, 7, 9, 13, 6, 3, 14, 1, 8],
     [5, 7, 10, 0, 1, 4, 2, 13, 12, 6, 3, 8, 11, 14, 15, 9],
     [15, 4, 5, 3, 2, 10, 11, 12, 7, 6, 0, 14, 13, 1, 9, 8],
     [6, 15, 13, 5, 11, 1, 9, 3, 2, 14, 7, 10, 8, 4, 12, 0],
     [8, 3, 1, 9, 13, 7, 12, 15, 2, 4, 0, 10, 11, 5, 6, 14],
     [11, 1, 8, 13, 7, 6, 14, 0, 10, 15, 5, 3, 12, 4, 9, 2],
     [5, 9, 13, 0, 2, 11, 10, 14, 8, 7, 1, 3, 4, 15, 6, 12],
     [11, 2, 12, 8, 3, 10, 13, 5, 4, 15, 0, 9, 14, 7, 6, 1],
     [2, 9, 11, 6, 8, 4, 7, 13, 15, 1, 5, 3, 0, 14, 12, 10],
     [5, 8, 6, 4, 12, 11, 14, 3, 0, 2, 1, 9, 7, 15, 10, 13],
     [15, 8, 9, 2, 11, 7, 14, 12, 0, 6, 1, 3, 13, 10, 4, 5],
     [1, 0, 13, 5, 14, 2, 10, 9, 15, 11, 8, 3, 6, 7, 4, 12],
     [12, 13, 9, 15, 6, 10, 3, 8, 0, 5, 7, 4, 14, 11, 2, 1],
     [0, 12, 5, 10, 15, 11, 9, 2, 1, 7, 4, 3, 6, 14, 8, 13],
     [13, 2, 8, 6, 3, 10, 0, 9, 7, 11, 4, 14, 12, 15, 1, 5]],
    dtype=np.int32,
)
_INV_NP = np.argsort(_IDXS_NP, axis=1).astype(np.int32)

_CB = 16


def _scatter_body(inv_ref, in_ref, out_ref):
    r = pl.program_id(0)
    N = in_ref.shape[0]
    Wc = in_ref.shape[3] // _CCHUNK
    for m in range(N):
        for c in range(_CCHUNK):
            n = inv_ref[r * _CCHUNK + c, m]
            out_ref[pl.ds(n, 1), :, :, pl.ds(c * Wc, Wc)] = in_ref[
                pl.ds(m, 1), :, :, pl.ds(c * Wc, Wc)
            ]


def kernel(protos):
    N, C, H, W = protos.shape
    Hc = H // _RCHUNK

    def in_map(r, cb, inv_ref):
        return (0, cb, r, 0)

    def out_map(r, cb, inv_ref):
        return (0, cb, r, 0)

    spro = pl.pallas_call(
        _scatter_body,
        grid_spec=pltpu.PrefetchScalarGridSpec(
            num_scalar_prefetch=1,
            grid=(_RCHUNK, C // _CB),
            in_specs=[pl.BlockSpec((N, _CB, Hc, W), in_map)],
            out_specs=pl.BlockSpec((N, _CB, Hc, W), out_map),
        ),
        out_shape=jax.ShapeDtypeStruct((N, C, H, W), protos.dtype),
    )(jnp.asarray(_INV_NP), protos)
    return spro, jnp.asarray(_IDXS_NP)
